# baseline (speedup 1.0000x reference)
import functools

import jax
import jax.numpy as jnp
from jax import lax
from jax.experimental import pallas as pl
from jax.experimental.pallas import tpu as pltpu
from jax.experimental.pallas import tpu_sc as plsc

N_NODES = 10000
N_EDGES = 320000
D = 128

NC = 2
NS = 16
NW = NC * NS
EPW = N_EDGES // NW
K = 125
NCH = EPW // K
PARTS = 2
NCH_P = NCH // PARTS
RPT = 624
R_REM = N_NODES - NS * RPT

BR = 2000


def _lin1_body(x_ref, w_ref, b_ref, o_ref):
    o_ref[...] = (
        jnp.dot(x_ref[...], w_ref[...], preferred_element_type=jnp.float32)
        + b_ref[...]
    )


def _lin2_body(p_ref, w_ref, b_ref, o_ref):
    h = jnp.maximum(p_ref[0] + p_ref[1], 0.0)
    o_ref[...] = (
        jnp.dot(h, w_ref[...], preferred_element_type=jnp.float32) + b_ref[...]
    )


def _lsm_body(p_ref, o_ref):
    z = p_ref[0] + p_ref[1]
    m = jnp.max(z, axis=-1, keepdims=True)
    e = jnp.exp(z - m)
    s = jnp.sum(e, axis=-1, keepdims=True)
    o_ref[...] = z - m - jnp.log(s)


def _lin1(x, w1t, b1):
    return pl.pallas_call(
        _lin1_body,
        grid=(N_NODES // BR,),
        in_specs=[
            pl.BlockSpec((BR, D), lambda i: (i, 0)),
            pl.BlockSpec((D, D), lambda i: (0, 0)),
            pl.BlockSpec((1, D), lambda i: (0, 0)),
        ],
        out_specs=pl.BlockSpec((BR, D), lambda i: (i, 0)),
        out_shape=jax.ShapeDtypeStruct((N_NODES, D), jnp.float32),
    )(x, w1t, b1.reshape(1, D))


def _lin2(p, w2t, b2):
    return pl.pallas_call(
        _lin2_body,
        grid=(N_NODES // BR,),
        in_specs=[
            pl.BlockSpec((NC, BR, D), lambda i: (0, i, 0)),
            pl.BlockSpec((D, D), lambda i: (0, 0)),
            pl.BlockSpec((1, D), lambda i: (0, 0)),
        ],
        out_specs=pl.BlockSpec((BR, D), lambda i: (i, 0)),
        out_shape=jax.ShapeDtypeStruct((N_NODES, D), jnp.float32),
    )(p, w2t, b2.reshape(1, D))


def _log_softmax(p):
    return pl.pallas_call(
        _lsm_body,
        grid=(N_NODES // BR,),
        in_specs=[pl.BlockSpec((NC, BR, D), lambda i: (0, i, 0))],
        out_specs=pl.BlockSpec((BR, D), lambda i: (i, 0)),
        out_shape=jax.ShapeDtypeStruct((N_NODES, D), jnp.float32),
    )(p)


def _spmm_body(h_hbm, row_hbm, col_hbm, zeros_hbm, out_hbm,
               col_v, row_v, bufs, gsems, ssems, accum):
    c = lax.axis_index("c")
    s = lax.axis_index("s")
    w = c * NS + s

    pltpu.sync_copy(
        zeros_hbm.at[pl.ds(s * RPT, RPT)],
        accum.at[pl.ds(s * RPT, RPT)],
    )

    @pl.when(s == NS - 1)
    def _zero_rem():
        pltpu.sync_copy(
            zeros_hbm.at[pl.ds(NS * RPT, R_REM)],
            accum.at[pl.ds(NS * RPT, R_REM)],
        )

    plsc.subcore_barrier()

    for part in range(PARTS):
        pltpu.sync_copy(col_hbm.at[w, part], col_v)
        pltpu.sync_copy(row_hbm.at[w, part], row_v)

        pltpu.async_copy(h_hbm.at[col_v.at[0]], bufs[0], gsems[0])

        @pl.loop(0, NCH_P, step=2)
        def _chunk(j):
            for p in range(2):
                ch = j + p
                q = 1 - p
                pltpu.make_async_copy(
                    h_hbm.at[col_v.at[ch]], bufs[p], gsems[p]
                ).wait()
                pltpu.async_copy(
                    bufs[p], accum.at[row_v.at[ch]], ssems[p], add=True
                )

                @pl.when(ch >= 1)
                def _drain_prev():
                    pltpu.make_async_copy(
                        bufs[q], accum.at[row_v.at[ch - 1]], ssems[q]
                    ).wait()

                @pl.when(ch + 1 < NCH_P)
                def _refill():
                    pltpu.async_copy(
                        h_hbm.at[col_v.at[ch + 1]], bufs[q], gsems[q]
                    )

        pltpu.make_async_copy(
            bufs[1], accum.at[row_v.at[NCH_P - 1]], ssems[1]
        ).wait()

    plsc.subcore_barrier()
    pltpu.sync_copy(
        accum.at[pl.ds(s * RPT, RPT)],
        out_hbm.at[c, pl.ds(s * RPT, RPT)],
    )

    @pl.when(s == NS - 1)
    def _wb_rem():
        pltpu.sync_copy(
            accum.at[pl.ds(NS * RPT, R_REM)],
            out_hbm.at[c, pl.ds(NS * RPT, R_REM)],
        )


_spmm = pl.kernel(
    _spmm_body,
    out_type=jax.ShapeDtypeStruct((NC, N_NODES, D), jnp.float32),
    mesh=plsc.VectorSubcoreMesh(
        core_axis_name="c", subcore_axis_name="s", num_cores=NC, num_subcores=NS
    ),
    scratch_types=[
        pltpu.VMEM((NCH_P, K), jnp.int32),
        pltpu.VMEM((NCH_P, K), jnp.int32),
        [pltpu.VMEM((K, D), jnp.float32) for _ in range(2)],
        [pltpu.SemaphoreType.DMA for _ in range(2)],
        [pltpu.SemaphoreType.DMA for _ in range(2)],
        pltpu.VMEM_SHARED((N_NODES, D), jnp.float32),
    ],
)


def kernel(x, edge_index, W1, b1, W2, b2):
    row = edge_index[0].astype(jnp.int32).reshape(NW, PARTS, NCH_P, K)
    col = edge_index[1].astype(jnp.int32).reshape(NW, PARTS, NCH_P, K)
    zeros = jnp.zeros((N_NODES, D), jnp.float32)

    h = _lin1(x, W1.T, b1)
    p1 = _spmm(h, row, col, zeros)
    h2 = _lin2(p1, W2.T, b2)
    p2 = _spmm(h2, row, col, zeros)
    return _log_softmax(p2)


# speedup vs baseline: 1.1616x; 1.1616x over previous
import functools

import jax
import jax.numpy as jnp
from jax import lax
from jax.experimental import pallas as pl
from jax.experimental.pallas import tpu as pltpu
from jax.experimental.pallas import tpu_sc as plsc

N_NODES = 10000
N_EDGES = 320000
D = 128

NC = 2
NS = 16
NW = NC * NS
EPW = N_EDGES // NW
K = 125
NCH = EPW // K
PARTS = 2
NCH_P = NCH // PARTS
RPT = 624
R_REM = N_NODES - NS * RPT

BR = 2000


def _lin1_body(x_ref, w_ref, b_ref, o_ref):
    o_ref[...] = (
        jnp.dot(x_ref[...], w_ref[...], preferred_element_type=jnp.float32)
        + b_ref[...]
    )


def _lin2_body(p_ref, w_ref, b_ref, o_ref):
    h = jnp.maximum(p_ref[0] + p_ref[1], 0.0)
    o_ref[...] = (
        jnp.dot(h, w_ref[...], preferred_element_type=jnp.float32) + b_ref[...]
    )


def _lsm_body(p_ref, o_ref):
    z = p_ref[0] + p_ref[1]
    m = jnp.max(z, axis=-1, keepdims=True)
    e = jnp.exp(z - m)
    s = jnp.sum(e, axis=-1, keepdims=True)
    o_ref[...] = z - m - jnp.log(s)


def _lin1(x, w1t, b1):
    return pl.pallas_call(
        _lin1_body,
        grid=(N_NODES // BR,),
        in_specs=[
            pl.BlockSpec((BR, D), lambda i: (i, 0)),
            pl.BlockSpec((D, D), lambda i: (0, 0)),
            pl.BlockSpec((1, D), lambda i: (0, 0)),
        ],
        out_specs=pl.BlockSpec((BR, D), lambda i: (i, 0)),
        out_shape=jax.ShapeDtypeStruct((N_NODES, D), jnp.float32),
    )(x, w1t, b1.reshape(1, D))


def _lin2(p, w2t, b2):
    return pl.pallas_call(
        _lin2_body,
        grid=(N_NODES // BR,),
        in_specs=[
            pl.BlockSpec((NC, BR, D), lambda i: (0, i, 0)),
            pl.BlockSpec((D, D), lambda i: (0, 0)),
            pl.BlockSpec((1, D), lambda i: (0, 0)),
        ],
        out_specs=pl.BlockSpec((BR, D), lambda i: (i, 0)),
        out_shape=jax.ShapeDtypeStruct((N_NODES, D), jnp.float32),
    )(p, w2t, b2.reshape(1, D))


def _log_softmax(p):
    return pl.pallas_call(
        _lsm_body,
        grid=(N_NODES // BR,),
        in_specs=[pl.BlockSpec((NC, BR, D), lambda i: (0, i, 0))],
        out_specs=pl.BlockSpec((BR, D), lambda i: (i, 0)),
        out_shape=jax.ShapeDtypeStruct((N_NODES, D), jnp.float32),
    )(p)


def _spmm_body(h_hbm, row_hbm, col_hbm, zeros_hbm, out_hbm,
               col_v, row_v, bufs, gsems, ssems, accum):
    c = lax.axis_index("c")
    s = lax.axis_index("s")
    w = c * NS + s

    pltpu.sync_copy(
        zeros_hbm.at[pl.ds(s * RPT, RPT)],
        accum.at[pl.ds(s * RPT, RPT)],
    )

    @pl.when(s == NS - 1)
    def _zero_rem():
        pltpu.sync_copy(
            zeros_hbm.at[pl.ds(NS * RPT, R_REM)],
            accum.at[pl.ds(NS * RPT, R_REM)],
        )

    plsc.subcore_barrier()

    for part in range(PARTS):
        pltpu.sync_copy(col_hbm.at[w, part], col_v)
        pltpu.sync_copy(row_hbm.at[w, part], row_v)

        pltpu.async_copy(h_hbm.at[col_v.at[0]], bufs[0], gsems[0])
        pltpu.async_copy(h_hbm.at[col_v.at[1]], bufs[1], gsems[1])

        @pl.loop(0, NCH_P, step=2)
        def _chunk(j):
            for p in range(2):
                ch = j + p
                pltpu.make_async_copy(
                    h_hbm.at[col_v.at[ch]], bufs[p], gsems[p]
                ).wait()
                pltpu.sync_copy(bufs[p], accum.at[row_v.at[ch]], add=True)

                @pl.when(ch + 2 < NCH_P)
                def _refill():
                    pltpu.async_copy(
                        h_hbm.at[col_v.at[ch + 2]], bufs[p], gsems[p]
                    )

    plsc.subcore_barrier()
    pltpu.sync_copy(
        accum.at[pl.ds(s * RPT, RPT)],
        out_hbm.at[c, pl.ds(s * RPT, RPT)],
    )

    @pl.when(s == NS - 1)
    def _wb_rem():
        pltpu.sync_copy(
            accum.at[pl.ds(NS * RPT, R_REM)],
            out_hbm.at[c, pl.ds(NS * RPT, R_REM)],
        )


_spmm = pl.kernel(
    _spmm_body,
    out_type=jax.ShapeDtypeStruct((NC, N_NODES, D), jnp.float32),
    mesh=plsc.VectorSubcoreMesh(
        core_axis_name="c", subcore_axis_name="s", num_cores=NC, num_subcores=NS
    ),
    scratch_types=[
        pltpu.VMEM((NCH_P, K), jnp.int32),
        pltpu.VMEM((NCH_P, K), jnp.int32),
        [pltpu.VMEM((K, D), jnp.float32) for _ in range(2)],
        [pltpu.SemaphoreType.DMA for _ in range(2)],
        [pltpu.SemaphoreType.DMA for _ in range(2)],
        pltpu.VMEM_SHARED((N_NODES, D), jnp.float32),
    ],
)


def kernel(x, edge_index, W1, b1, W2, b2):
    row = edge_index[0].astype(jnp.int32).reshape(NW, PARTS, NCH_P, K)
    col = edge_index[1].astype(jnp.int32).reshape(NW, PARTS, NCH_P, K)
    zeros = jnp.zeros((N_NODES, D), jnp.float32)

    h = _lin1(x, W1.T, b1)
    p1 = _spmm(h, row, col, zeros)
    h2 = _lin2(p1, W2.T, b2)
    p2 = _spmm(h2, row, col, zeros)
    return _log_softmax(p2)


# speedup vs baseline: 1.1767x; 1.0130x over previous
import functools

import jax
import jax.numpy as jnp
from jax import lax
from jax.experimental import pallas as pl
from jax.experimental.pallas import tpu as pltpu
from jax.experimental.pallas import tpu_sc as plsc

N_NODES = 10000
N_EDGES = 320000
D = 128

NC = 2
NS = 16
NW = NC * NS
EPW = N_EDGES // NW
K = 125
NCH = EPW // K
PARTS = 2
NCH_P = NCH // PARTS
RPT = 624
R_REM = N_NODES - NS * RPT

BR = 2000


def _lin1_body(x_ref, w_ref, b_ref, o_ref):
    o_ref[...] = (
        jnp.dot(x_ref[...], w_ref[...], preferred_element_type=jnp.float32)
        + b_ref[...]
    )


def _lin2_body(p_ref, w_ref, b_ref, o_ref):
    h = jnp.maximum(p_ref[0] + p_ref[1], 0.0)
    o_ref[...] = (
        jnp.dot(h, w_ref[...], preferred_element_type=jnp.float32) + b_ref[...]
    )


def _lsm_body(p_ref, o_ref):
    z = p_ref[0] + p_ref[1]
    m = jnp.max(z, axis=-1, keepdims=True)
    e = jnp.exp(z - m)
    s = jnp.sum(e, axis=-1, keepdims=True)
    o_ref[...] = z - m - jnp.log(s)


def _lin1(x, w1t, b1):
    return pl.pallas_call(
        _lin1_body,
        grid=(N_NODES // BR,),
        in_specs=[
            pl.BlockSpec((BR, D), lambda i: (i, 0)),
            pl.BlockSpec((D, D), lambda i: (0, 0)),
            pl.BlockSpec((1, D), lambda i: (0, 0)),
        ],
        out_specs=pl.BlockSpec((BR, D), lambda i: (i, 0)),
        out_shape=jax.ShapeDtypeStruct((N_NODES, D), jnp.float32),
    )(x, w1t, b1.reshape(1, D))


def _lin2(p, w2t, b2):
    return pl.pallas_call(
        _lin2_body,
        grid=(N_NODES // BR,),
        in_specs=[
            pl.BlockSpec((NC, BR, D), lambda i: (0, i, 0)),
            pl.BlockSpec((D, D), lambda i: (0, 0)),
            pl.BlockSpec((1, D), lambda i: (0, 0)),
        ],
        out_specs=pl.BlockSpec((BR, D), lambda i: (i, 0)),
        out_shape=jax.ShapeDtypeStruct((N_NODES, D), jnp.float32),
    )(p, w2t, b2.reshape(1, D))


def _log_softmax(p):
    return pl.pallas_call(
        _lsm_body,
        grid=(N_NODES // BR,),
        in_specs=[pl.BlockSpec((NC, BR, D), lambda i: (0, i, 0))],
        out_specs=pl.BlockSpec((BR, D), lambda i: (i, 0)),
        out_shape=jax.ShapeDtypeStruct((N_NODES, D), jnp.float32),
    )(p)


def _spmm_body(h_hbm, row_hbm, col_hbm, zeros_hbm, out_hbm,
               col_v, row_v, bufs, gsems, ssems, accum):
    c = lax.axis_index("c")
    s = lax.axis_index("s")
    w = c * NS + s

    pltpu.sync_copy(col_hbm.at[w, 0], col_v)
    pltpu.sync_copy(row_hbm.at[w, 0], row_v)
    pltpu.async_copy(h_hbm.at[col_v.at[0]], bufs[0], gsems[0])
    pltpu.async_copy(h_hbm.at[col_v.at[1]], bufs[1], gsems[1])

    pltpu.sync_copy(
        zeros_hbm.at[pl.ds(s * RPT, RPT)],
        accum.at[pl.ds(s * RPT, RPT)],
    )

    @pl.when(s == NS - 1)
    def _zero_rem():
        pltpu.sync_copy(
            zeros_hbm.at[pl.ds(NS * RPT, R_REM)],
            accum.at[pl.ds(NS * RPT, R_REM)],
        )

    plsc.subcore_barrier()

    for part in range(PARTS):
        if part > 0:
            pltpu.sync_copy(col_hbm.at[w, part], col_v)
            pltpu.sync_copy(row_hbm.at[w, part], row_v)

            pltpu.async_copy(h_hbm.at[col_v.at[0]], bufs[0], gsems[0])
            pltpu.async_copy(h_hbm.at[col_v.at[1]], bufs[1], gsems[1])

        @pl.loop(0, NCH_P, step=2)
        def _chunk(j):
            for p in range(2):
                ch = j + p
                pltpu.make_async_copy(
                    h_hbm.at[col_v.at[ch]], bufs[p], gsems[p]
                ).wait()
                pltpu.sync_copy(bufs[p], accum.at[row_v.at[ch]], add=True)

                @pl.when(ch + 2 < NCH_P)
                def _refill():
                    pltpu.async_copy(
                        h_hbm.at[col_v.at[ch + 2]], bufs[p], gsems[p]
                    )

    plsc.subcore_barrier()
    pltpu.sync_copy(
        accum.at[pl.ds(s * RPT, RPT)],
        out_hbm.at[c, pl.ds(s * RPT, RPT)],
    )

    @pl.when(s == NS - 1)
    def _wb_rem():
        pltpu.sync_copy(
            accum.at[pl.ds(NS * RPT, R_REM)],
            out_hbm.at[c, pl.ds(NS * RPT, R_REM)],
        )


_spmm = pl.kernel(
    _spmm_body,
    out_type=jax.ShapeDtypeStruct((NC, N_NODES, D), jnp.float32),
    mesh=plsc.VectorSubcoreMesh(
        core_axis_name="c", subcore_axis_name="s", num_cores=NC, num_subcores=NS
    ),
    scratch_types=[
        pltpu.VMEM((NCH_P, K), jnp.int32),
        pltpu.VMEM((NCH_P, K), jnp.int32),
        [pltpu.VMEM((K, D), jnp.float32) for _ in range(2)],
        [pltpu.SemaphoreType.DMA for _ in range(2)],
        [pltpu.SemaphoreType.DMA for _ in range(2)],
        pltpu.VMEM_SHARED((N_NODES, D), jnp.float32),
    ],
)


def kernel(x, edge_index, W1, b1, W2, b2):
    row = edge_index[0].astype(jnp.int32).reshape(NW, PARTS, NCH_P, K)
    col = edge_index[1].astype(jnp.int32).reshape(NW, PARTS, NCH_P, K)
    zeros = jnp.zeros((N_NODES, D), jnp.float32)

    h = _lin1(x, W1.T, b1)
    p1 = _spmm(h, row, col, zeros)
    h2 = _lin2(p1, W2.T, b2)
    p2 = _spmm(h2, row, col, zeros)
    return _log_softmax(p2)


# speedup vs baseline: 1.1789x; 1.0019x over previous
import jax
import jax.numpy as jnp
from jax import lax
from jax.experimental import pallas as pl
from jax.experimental.pallas import tpu as pltpu
from jax.experimental.pallas import tpu_sc as plsc

N_NODES = 10000
N_EDGES = 320000
D = 128

NC = 2
NS = 16
NW = NC * NS
EPW = N_EDGES // NW
K = 125
NCH = EPW // K
PARTS = 2
NCH_P = NCH // PARTS
RPT = 624
R_REM = N_NODES - NS * RPT

BR = 2000


def _lin1_body(x_ref, w_ref, b_ref, o_ref):
    o_ref[...] = (
        jnp.dot(x_ref[...], w_ref[...], preferred_element_type=jnp.float32)
        + b_ref[...]
    )


def _lin2_body(p_ref, w_ref, b_ref, o_ref):
    h = jnp.maximum(p_ref[0] + p_ref[1], 0.0)
    o_ref[...] = (
        jnp.dot(h, w_ref[...], preferred_element_type=jnp.float32) + b_ref[...]
    )


def _lsm_body(p_ref, o_ref):
    z = p_ref[0] + p_ref[1]
    m = jnp.max(z, axis=-1, keepdims=True)
    e = jnp.exp(z - m)
    s = jnp.sum(e, axis=-1, keepdims=True)
    o_ref[...] = z - m - jnp.log(s)


def _lin1(x, w1t, b1):
    return pl.pallas_call(
        _lin1_body,
        grid=(N_NODES // BR,),
        in_specs=[
            pl.BlockSpec((BR, D), lambda i: (i, 0)),
            pl.BlockSpec((D, D), lambda i: (0, 0)),
            pl.BlockSpec((1, D), lambda i: (0, 0)),
        ],
        out_specs=pl.BlockSpec((BR, D), lambda i: (i, 0)),
        out_shape=jax.ShapeDtypeStruct((N_NODES, D), jnp.float32),
    )(x, w1t, b1.reshape(1, D))


def _lin2(p, w2t, b2):
    return pl.pallas_call(
        _lin2_body,
        grid=(N_NODES // BR,),
        in_specs=[
            pl.BlockSpec((NC, BR, D), lambda i: (0, i, 0)),
            pl.BlockSpec((D, D), lambda i: (0, 0)),
            pl.BlockSpec((1, D), lambda i: (0, 0)),
        ],
        out_specs=pl.BlockSpec((BR, D), lambda i: (i, 0)),
        out_shape=jax.ShapeDtypeStruct((N_NODES, D), jnp.float32),
    )(p, w2t, b2.reshape(1, D))


def _log_softmax(p):
    return pl.pallas_call(
        _lsm_body,
        grid=(N_NODES // BR,),
        in_specs=[pl.BlockSpec((NC, BR, D), lambda i: (0, i, 0))],
        out_specs=pl.BlockSpec((BR, D), lambda i: (i, 0)),
        out_shape=jax.ShapeDtypeStruct((N_NODES, D), jnp.float32),
    )(p)


def _spmm_body(h_hbm, row_hbm, col_hbm, zeros_hbm, out_hbm,
               col_v, row_v, bufs, gsems, ssems, accum):
    c = lax.axis_index("c")
    s = lax.axis_index("s")
    w = c * NS + s

    pltpu.sync_copy(col_hbm.at[w, 0], col_v)
    pltpu.sync_copy(row_hbm.at[w, 0], row_v)
    pltpu.async_copy(h_hbm.at[col_v.at[0]], bufs[0], gsems[0])
    pltpu.async_copy(h_hbm.at[col_v.at[1]], bufs[1], gsems[1])

    pltpu.sync_copy(
        zeros_hbm.at[pl.ds(s * RPT, RPT)],
        accum.at[pl.ds(s * RPT, RPT)],
    )

    @pl.when(s == NS - 1)
    def _zero_rem():
        pltpu.sync_copy(
            zeros_hbm.at[pl.ds(NS * RPT, R_REM)],
            accum.at[pl.ds(NS * RPT, R_REM)],
        )

    plsc.subcore_barrier()

    for part in range(PARTS):
        if part > 0:
            pltpu.sync_copy(col_hbm.at[w, part], col_v)
            pltpu.sync_copy(row_hbm.at[w, part], row_v)

            pltpu.async_copy(h_hbm.at[col_v.at[0]], bufs[0], gsems[0])
            pltpu.async_copy(h_hbm.at[col_v.at[1]], bufs[1], gsems[1])

        @pl.loop(0, NCH_P, step=2)
        def _chunk(j):
            for p in range(2):
                ch = j + p
                pltpu.make_async_copy(
                    h_hbm.at[col_v.at[ch]], bufs[p], gsems[p]
                ).wait()
                pltpu.sync_copy(bufs[p], accum.at[row_v.at[ch]], add=True)

                @pl.when(ch + 2 < NCH_P)
                def _refill():
                    pltpu.async_copy(
                        h_hbm.at[col_v.at[ch + 2]], bufs[p], gsems[p]
                    )

    plsc.subcore_barrier()
    pltpu.sync_copy(
        accum.at[pl.ds(s * RPT, RPT)],
        out_hbm.at[c, pl.ds(s * RPT, RPT)],
    )

    @pl.when(s == NS - 1)
    def _wb_rem():
        pltpu.sync_copy(
            accum.at[pl.ds(NS * RPT, R_REM)],
            out_hbm.at[c, pl.ds(NS * RPT, R_REM)],
        )


_spmm = pl.kernel(
    _spmm_body,
    out_type=jax.ShapeDtypeStruct((NC, N_NODES, D), jnp.float32),
    mesh=plsc.VectorSubcoreMesh(
        core_axis_name="c", subcore_axis_name="s", num_cores=NC, num_subcores=NS
    ),
    scratch_types=[
        pltpu.VMEM((NCH_P, K), jnp.int32),
        pltpu.VMEM((NCH_P, K), jnp.int32),
        [pltpu.VMEM((K, D), jnp.float32) for _ in range(2)],
        [pltpu.SemaphoreType.DMA for _ in range(2)],
        [pltpu.SemaphoreType.DMA for _ in range(2)],
        pltpu.VMEM_SHARED((N_NODES, D), jnp.float32),
    ],
)


def kernel(x, edge_index, W1, b1, W2, b2):
    row = edge_index[0].astype(jnp.int32).reshape(NW, PARTS, NCH_P, K)
    col = edge_index[1].astype(jnp.int32).reshape(NW, PARTS, NCH_P, K)
    zeros = jnp.zeros((N_NODES, D), jnp.float32)

    h = _lin1(x, W1.T, b1)
    p1 = _spmm(h, row, col, zeros)
    h2 = _lin2(p1, W2.T, b2)
    p2 = _spmm(h2, row, col, zeros)
    return _log_softmax(p2)
